# baseline (speedup 1.0000x reference)
import functools

import jax
import jax.numpy as jnp
from jax import lax
from jax.experimental import pallas as pl
from jax.experimental.pallas import tpu as pltpu
from jax.experimental.pallas import tpu_sc as plsc

B = 256
F = 128
L = 16


def _fm_body(user_hbm, item_hbm, eu_hbm, ei_hbm, ub_hbm, ib_hbm, bias_hbm,
             out_hbm, yfm_hbm, uidx_v, iidx_v, eu_v, ei_v, ub_v, ib_v, bias_v,
             yv_v, stage_v, out_v, sem):
    c = lax.axis_index("c")
    s = lax.axis_index("s")
    base = s * L

    pltpu.sync_copy(user_hbm.at[pl.ds(base, L)], uidx_v)
    pltpu.sync_copy(item_hbm.at[pl.ds(base, L)], iidx_v)
    eu_cp = pltpu.async_copy(eu_hbm.at[uidx_v], eu_v, sem)
    ei_cp = pltpu.async_copy(ei_hbm.at[iidx_v], ei_v, sem)
    ub_cp = pltpu.async_copy(ub_hbm.at[uidx_v], ub_v, sem)
    ib_cp = pltpu.async_copy(ib_hbm.at[iidx_v], ib_v, sem)
    pltpu.sync_copy(bias_hbm, bias_v)
    eu_cp.wait()
    ei_cp.wait()
    ub_cp.wait()
    ib_cp.wait()

    iota = lax.iota(jnp.int32, L)

    yv = jnp.zeros((L,), jnp.float32)
    for f in range(F):
        fcol = jnp.full((L,), f, jnp.int32)
        fu = plsc.load_gather(eu_v, [iota, fcol])
        fi = plsc.load_gather(ei_v, [iota, fcol])
        yv = yv + fu * fi
    yv_v[...] = yv

    pltpu.sync_copy(yv_v, yfm_hbm.at[c, s])
    plsc.subcore_barrier()
    pltpu.sync_copy(yfm_hbm.at[c], stage_v)

    for r in range(8):
        lr = c * 8 + r
        lane = jnp.full((L,), lr, jnp.int32)
        csplat = (plsc.load_gather(ub_v, [lane])
                  + plsc.load_gather(ib_v, [lane]) + bias_v[...])
        for cc in range(B // L):
            yfm_cc = stage_v[cc, pl.ds(0, L)]
            if cc < F // L:
                emb = eu_v[lr, pl.ds(cc * L, L)]
            else:
                emb = ei_v[lr, pl.ds((cc - F // L) * L, L)]
            out_v[r, pl.ds(cc * L, L)] = emb + yfm_cc + csplat
    pltpu.sync_copy(out_v, out_hbm.at[pl.ds(base + c * 8, 8)])


@functools.partial(jax.jit, static_argnames=())
def _fm_call(user, item, embed_user, embed_item, ub1, ib1, b16):
    mesh = plsc.VectorSubcoreMesh(core_axis_name="c", subcore_axis_name="s")
    run = pl.kernel(
        _fm_body,
        out_type=(jax.ShapeDtypeStruct((B, B), jnp.float32),
                  jax.ShapeDtypeStruct((2, L, L), jnp.float32)),
        mesh=mesh,
        compiler_params=pltpu.CompilerParams(needs_layout_passes=False),
        scratch_types=[
            pltpu.VMEM((L,), jnp.int32),
            pltpu.VMEM((L,), jnp.int32),
            pltpu.VMEM((L, F), jnp.float32),
            pltpu.VMEM((L, F), jnp.float32),
            pltpu.VMEM((L,), jnp.float32),
            pltpu.VMEM((L,), jnp.float32),
            pltpu.VMEM((L,), jnp.float32),
            pltpu.VMEM((L,), jnp.float32),
            pltpu.VMEM((L, L), jnp.float32),
            pltpu.VMEM((8, B), jnp.float32),
            pltpu.SemaphoreType.DMA,
        ],
    )
    out, _ = run(user, item, embed_user, embed_item, ub1, ib1, b16)
    return out


def kernel(user, item, embed_user, embed_item, u_bias, i_bias, bias_):
    user = user.astype(jnp.int32)
    item = item.astype(jnp.int32)
    ub1 = u_bias.reshape(-1)
    ib1 = i_bias.reshape(-1)
    b16 = jnp.broadcast_to(bias_, (L,))
    out = _fm_call(user, item, embed_user, embed_item, ub1, ib1, b16)
    return out.reshape(-1)
